# baseline (speedup 1.0000x reference)
import functools

import jax
import jax.numpy as jnp
from jax.experimental import pallas as pl
from jax.experimental.pallas import tpu as pltpu

_N = 1_000_000
_R, _C = 1000, 1000
_BR = 40
_GRID = _R // _BR
_EPS = 1e-12
_NEG_INF = float("-inf")
_IMAX = 2**31 - 1


def _body(x_ref, u_ref, act_ref, lp_ref, best_ref, idx_ref, blog_ref, s_ref):
    i = pl.program_id(0)
    x = x_ref[...]
    uu = u_ref[...]
    g = -jnp.log(-jnp.log(uu + _EPS) + _EPS)
    p = x + g
    r = jax.lax.broadcasted_iota(jnp.int32, (_BR, _C), 0)
    c = jax.lax.broadcasted_iota(jnp.int32, (_BR, _C), 1)
    idx = (i * _BR + r) * _C + c
    e = jnp.exp(x)

    @pl.when(i == 0)
    def _init():
        best_ref[...] = p
        idx_ref[...] = idx
        blog_ref[...] = x
        s_ref[...] = e

    @pl.when(i > 0)
    def _acc():
        m = p > best_ref[...]
        best_ref[...] = jnp.where(m, p, best_ref[...])
        idx_ref[...] = jnp.where(m, idx, idx_ref[...])
        blog_ref[...] = jnp.where(m, x, blog_ref[...])
        s_ref[...] = s_ref[...] + e

    @pl.when(i == _GRID - 1)
    def _fin():
        bv = best_ref[...]
        mx = jnp.max(bv)
        winners = bv == mx
        a = jnp.min(jnp.where(winners, idx_ref[...], _IMAX))
        bl = jnp.max(jnp.where(winners & (idx_ref[...] == a), blog_ref[...],
                               _NEG_INF))
        s = jnp.sum(s_ref[...])
        act_ref[0] = a
        lp_ref[0] = bl - jnp.log(s)


@functools.partial(jax.jit)
def kernel(logits, u):
    x2 = logits.reshape(_R, _C)
    u2 = u.reshape(_R, _C)
    act, lp = pl.pallas_call(
        _body,
        grid=(_GRID,),
        in_specs=[
            pl.BlockSpec((_BR, _C), lambda i: (i, 0)),
            pl.BlockSpec((_BR, _C), lambda i: (i, 0)),
        ],
        out_specs=[
            pl.BlockSpec(memory_space=pltpu.SMEM),
            pl.BlockSpec(memory_space=pltpu.SMEM),
        ],
        out_shape=[
            jax.ShapeDtypeStruct((1,), jnp.int32),
            jax.ShapeDtypeStruct((1,), jnp.float32),
        ],
        scratch_shapes=[
            pltpu.VMEM((_BR, _C), jnp.float32),
            pltpu.VMEM((_BR, _C), jnp.int32),
            pltpu.VMEM((_BR, _C), jnp.float32),
            pltpu.VMEM((_BR, _C), jnp.float32),
        ],
    )(x2, u2)
    return act[0], lp[0]
